# baseline (speedup 1.0000x reference)
import functools

import jax
import jax.numpy as jnp
from jax import lax
from jax.experimental import pallas as pl
from jax.experimental.pallas import tpu as pltpu
from jax.experimental.pallas import tpu_sc as plsc

EMBED_DIM = 64
NUM_CORES = 2
NUM_SUBCORES = 16
NUM_WORKERS = NUM_CORES * NUM_SUBCORES
NBUF = 2
LANES = 16


def _make_call(batch, hist):
    bw = batch // NUM_WORKERS
    assert batch % NUM_WORKERS == 0
    mesh = plsc.VectorSubcoreMesh(core_axis_name="c", subcore_axis_name="s")

    @functools.partial(
        pl.kernel,
        mesh=mesh,
        compiler_params=pltpu.CompilerParams(
            use_tc_tiling_on_sc=False, needs_layout_passes=False),
        out_type=jax.ShapeDtypeStruct((hist, EMBED_DIM, batch), jnp.float32),
        scratch_types=[
            pltpu.VMEM((NBUF, bw), jnp.int32),
            pltpu.VMEM((bw, EMBED_DIM), jnp.float32),
            pltpu.VMEM((NBUF, EMBED_DIM, bw), jnp.float32),
            pltpu.SemaphoreType.DMA,
            pltpu.SemaphoreType.DMA,
            pltpu.SemaphoreType.DMA,
        ],
    )
    def gather_kernel(table_hbm, xt_hbm, out_hbm, idx_v, rows_v, trows_v,
                      idx_sem, gat_sem, out_sem):
        wid = lax.axis_index("s") * NUM_CORES + lax.axis_index("c")
        b0 = wid * bw
        lane = lax.iota(jnp.int32, LANES)

        for b in range(NBUF):
            pltpu.async_copy(xt_hbm.at[b, pl.ds(b0, bw)], idx_v.at[b],
                             idx_sem)

        def outer(g, carry):
            for b in range(NBUF):
                h = g * NBUF + b
                pltpu.make_async_copy(
                    xt_hbm.at[0, pl.ds(b0, bw)], idx_v.at[b], idx_sem).wait()

                pltpu.async_copy(
                    table_hbm.at[idx_v.at[b]], rows_v, gat_sem).wait()

                @pl.when(g < (hist // NBUF) - 1)
                def _():
                    pltpu.async_copy(
                        xt_hbm.at[h + NBUF, pl.ds(b0, bw)], idx_v.at[b],
                        idx_sem)

                @pl.when(g > 0)
                def _():
                    pltpu.make_async_copy(
                        trows_v.at[b], out_hbm.at[0, :, pl.ds(b0, bw)],
                        out_sem).wait()

                def trans(r0, carry2):
                    row_idx = r0 * LANES + lane
                    for d in range(EMBED_DIM):
                        col_idx = jnp.full((LANES,), d, jnp.int32)
                        v = plsc.load_gather(rows_v, [row_idx, col_idx])
                        trows_v[b, d, pl.ds(r0 * LANES, LANES)] = v
                    return carry2

                lax.fori_loop(0, bw // LANES, trans, 0)

                pltpu.async_copy(
                    trows_v.at[b], out_hbm.at[h, :, pl.ds(b0, bw)], out_sem)
            return carry

        lax.fori_loop(0, hist // NBUF, outer, 0)

        for b in range(NBUF):
            pltpu.make_async_copy(
                trows_v.at[b], out_hbm.at[0, :, pl.ds(b0, bw)],
                out_sem).wait()

    return gather_kernel


@jax.jit
def kernel(x, table):
    batch, hist = x.shape
    xt = x.T.astype(jnp.int32)
    out_t = _make_call(batch, hist)(table, xt)
    return out_t.transpose(2, 0, 1)


# speedup vs baseline: 1.7249x; 1.7249x over previous
import functools

import jax
import jax.numpy as jnp
from jax import lax
from jax.experimental import pallas as pl
from jax.experimental.pallas import tpu as pltpu
from jax.experimental.pallas import tpu_sc as plsc

EMBED_DIM = 64
NUM_CORES = 2
NUM_SUBCORES = 16
NUM_WORKERS = NUM_CORES * NUM_SUBCORES
NBUF = 2


def _make_call(batch, hist):
    bw = batch // NUM_WORKERS
    assert batch % NUM_WORKERS == 0
    mesh = plsc.VectorSubcoreMesh(core_axis_name="c", subcore_axis_name="s")

    @functools.partial(
        pl.kernel,
        mesh=mesh,
        compiler_params=pltpu.CompilerParams(use_tc_tiling_on_sc=False),
        out_type=jax.ShapeDtypeStruct((batch, hist, EMBED_DIM), jnp.float32),
        scratch_types=[
            pltpu.VMEM((NBUF, bw), jnp.int32),
            pltpu.VMEM((NBUF, bw, EMBED_DIM), jnp.float32),
            pltpu.SemaphoreType.DMA,
            pltpu.SemaphoreType.DMA,
            pltpu.SemaphoreType.DMA,
        ],
    )
    def gather_kernel(table_hbm, xt_hbm, out_hbm, idx_v, rows_v,
                      idx_sem, gat_sem, out_sem):
        wid = lax.axis_index("s") * NUM_CORES + lax.axis_index("c")
        b0 = wid * bw

        for b in range(NBUF):
            pltpu.async_copy(xt_hbm.at[b, pl.ds(b0, bw)], idx_v.at[b],
                             idx_sem)

        def outer(g, carry):
            for b in range(NBUF):
                h = g * NBUF + b
                pltpu.make_async_copy(
                    xt_hbm.at[0, pl.ds(b0, bw)], idx_v.at[b], idx_sem).wait()

                @pl.when(g > 0)
                def _():
                    pltpu.make_async_copy(
                        rows_v.at[b], out_hbm.at[pl.ds(b0, bw), 0],
                        out_sem).wait()

                pltpu.async_copy(
                    table_hbm.at[idx_v.at[b]], rows_v.at[b], gat_sem).wait()

                pltpu.async_copy(
                    rows_v.at[b], out_hbm.at[pl.ds(b0, bw), h], out_sem)

                @pl.when(g < (hist // NBUF) - 1)
                def _():
                    pltpu.async_copy(
                        xt_hbm.at[h + NBUF, pl.ds(b0, bw)], idx_v.at[b],
                        idx_sem)
            return carry

        lax.fori_loop(0, hist // NBUF, outer, 0)

        for b in range(NBUF):
            pltpu.make_async_copy(
                rows_v.at[b], out_hbm.at[pl.ds(b0, bw), 0], out_sem).wait()

    return gather_kernel


@jax.jit
def kernel(x, table):
    batch, hist = x.shape
    xt = x.T.astype(jnp.int32)
    return _make_call(batch, hist)(table, xt)
